# Initial kernel scaffold; baseline (speedup 1.0000x reference)
import jax
import jax.numpy as jnp
from jax.experimental import pallas as pl


def kernel(token_ids, table):
    raise NotImplementedError("write your pallas kernel here")


# speedup vs baseline: 1.4591x; 1.4591x over previous
import functools

import jax
import jax.numpy as jnp
from jax import lax
from jax.experimental import pallas as pl
from jax.experimental.pallas import tpu as pltpu
from jax.experimental.pallas import tpu_sc as plsc

NUM_CORES = 2
NUM_SUBCORES = 16
NUM_WORKERS = NUM_CORES * NUM_SUBCORES
CHUNK = 1024


@functools.partial(jax.jit, static_argnums=(2, 3))
def _gather_rows(idx, table, n, d):
    b_per_w = n // NUM_WORKERS
    n_chunks = b_per_w // CHUNK
    mesh = plsc.VectorSubcoreMesh(core_axis_name="c", subcore_axis_name="s")

    @functools.partial(
        pl.kernel,
        mesh=mesh,
        out_type=jax.ShapeDtypeStruct((n, d), jnp.float32),
        scratch_types=[
            pltpu.VMEM((CHUNK,), jnp.int32),
            pltpu.VMEM((CHUNK, d), jnp.float32),
            pltpu.SemaphoreType.DMA,
        ],
        compiler_params=pltpu.CompilerParams(use_tc_tiling_on_sc=False),
    )
    def k(table_hbm, idx_hbm, out_hbm, idx_v, rows_v, sem):
        wid = lax.axis_index("s") * NUM_CORES + lax.axis_index("c")
        base = wid * b_per_w

        def body(i, carry):
            start = base + i * CHUNK
            pltpu.sync_copy(idx_hbm.at[pl.ds(start, CHUNK)], idx_v)
            pltpu.async_copy(table_hbm.at[idx_v], rows_v, sem).wait()
            pltpu.sync_copy(rows_v, out_hbm.at[pl.ds(start, CHUNK)])
            return carry

        lax.fori_loop(0, n_chunks, body, 0)

    return k(table, idx)


def kernel(token_ids, table):
    b, s = token_ids.shape
    v, d = table.shape
    flat = token_ids.reshape(-1).astype(jnp.int32)
    out = _gather_rows(flat, table, flat.shape[0], d)
    return out.reshape(b, s, d)


# speedup vs baseline: 1.4922x; 1.0227x over previous
import functools

import jax
import jax.numpy as jnp
from jax import lax
from jax.experimental import pallas as pl
from jax.experimental.pallas import tpu as pltpu
from jax.experimental.pallas import tpu_sc as plsc

NUM_CORES = 2
NUM_SUBCORES = 16
NUM_WORKERS = NUM_CORES * NUM_SUBCORES
CHUNK = 1600


@functools.partial(jax.jit, static_argnums=(2, 3))
def _gather_rows(idx, table, n, d):
    b_per_w = n // NUM_WORKERS
    n_chunks = b_per_w // CHUNK
    mesh = plsc.VectorSubcoreMesh(core_axis_name="c", subcore_axis_name="s")

    @functools.partial(
        pl.kernel,
        mesh=mesh,
        out_type=jax.ShapeDtypeStruct((n, d), jnp.float32),
        scratch_types=[
            pltpu.VMEM((2, CHUNK), jnp.int32),
            pltpu.VMEM((2, CHUNK, d), jnp.float32),
            pltpu.SemaphoreType.DMA((2,)),
            pltpu.SemaphoreType.DMA((2,)),
        ],
        compiler_params=pltpu.CompilerParams(use_tc_tiling_on_sc=False),
    )
    def k(table_hbm, idx_hbm, out_hbm, idx_v, rows_v, sem_g, sem_o):
        wid = lax.axis_index("s") * NUM_CORES + lax.axis_index("c")
        base = wid * b_per_w

        def start_gather(c, b):
            pltpu.sync_copy(idx_hbm.at[pl.ds(base + c * CHUNK, CHUNK)],
                            idx_v.at[b])
            pltpu.make_async_copy(
                table_hbm.at[idx_v.at[b]], rows_v.at[b], sem_g.at[b]).start()

        def wait_gather(b):
            pltpu.make_async_copy(
                table_hbm.at[idx_v.at[b]], rows_v.at[b], sem_g.at[b]).wait()

        def start_out(c, b):
            pltpu.make_async_copy(
                rows_v.at[b], out_hbm.at[pl.ds(base + c * CHUNK, CHUNK)],
                sem_o.at[b]).start()

        def wait_out(c, b):
            pltpu.make_async_copy(
                rows_v.at[b], out_hbm.at[pl.ds(base + c * CHUNK, CHUNK)],
                sem_o.at[b]).wait()

        start_gather(0, 0)

        def body(c, carry):
            b = lax.rem(c, 2)
            pb = lax.rem(c - 1, 2)

            @pl.when(c >= 2)
            def _():
                wait_out(c - 2, b)

            start_gather(c, b)
            wait_gather(pb)
            start_out(c - 1, pb)
            return carry

        lax.fori_loop(1, n_chunks, body, 0)

        bl = (n_chunks - 1) % 2
        wait_gather(bl)
        start_out(n_chunks - 1, bl)
        wait_out(n_chunks - 2, 1 - bl)
        wait_out(n_chunks - 1, bl)

    return k(table, idx)


def kernel(token_ids, table):
    b, s = token_ids.shape
    v, d = table.shape
    flat = token_ids.reshape(-1).astype(jnp.int32)
    out = _gather_rows(flat, table, flat.shape[0], d)
    return out.reshape(b, s, d)


# speedup vs baseline: 1.4925x; 1.0002x over previous
import functools

import jax
import jax.numpy as jnp
from jax import lax
from jax.experimental import pallas as pl
from jax.experimental.pallas import tpu as pltpu
from jax.experimental.pallas import tpu_sc as plsc

NUM_CORES = 2
NUM_SUBCORES = 16
NUM_WORKERS = NUM_CORES * NUM_SUBCORES
CHUNK = 1600


@functools.partial(jax.jit, static_argnums=(2,))
def _gather_rows(idx, table, out_shape):
    n = idx.shape[0]
    bsz, seq, d = out_shape
    rows_per_chunk = CHUNK // seq
    b_per_w = n // NUM_WORKERS
    n_chunks = b_per_w // CHUNK
    mesh = plsc.VectorSubcoreMesh(core_axis_name="c", subcore_axis_name="s")

    @functools.partial(
        pl.kernel,
        mesh=mesh,
        out_type=jax.ShapeDtypeStruct((bsz, seq, d), jnp.float32),
        scratch_types=[
            pltpu.VMEM((2, CHUNK), jnp.int32),
            pltpu.VMEM((2, CHUNK, d), jnp.float32),
            pltpu.SemaphoreType.DMA((2,)),
            pltpu.SemaphoreType.DMA((2,)),
        ],
        compiler_params=pltpu.CompilerParams(use_tc_tiling_on_sc=False),
    )
    def k(table_hbm, idx_hbm, out_hbm, idx_v, rows_v, sem_g, sem_o):
        wid = lax.axis_index("s") * NUM_CORES + lax.axis_index("c")
        base = wid * b_per_w
        row_base = wid * (b_per_w // seq)

        def start_gather(c, b):
            pltpu.sync_copy(idx_hbm.at[pl.ds(base + c * CHUNK, CHUNK)],
                            idx_v.at[b])
            pltpu.make_async_copy(
                table_hbm.at[idx_v.at[b]], rows_v.at[b], sem_g.at[b]).start()

        def wait_gather(b):
            pltpu.make_async_copy(
                table_hbm.at[idx_v.at[b]], rows_v.at[b], sem_g.at[b]).wait()

        def start_out(c, b):
            for j in range(rows_per_chunk):
                pltpu.make_async_copy(
                    rows_v.at[b, pl.ds(j * seq, seq)],
                    out_hbm.at[row_base + c * rows_per_chunk + j],
                    sem_o.at[b]).start()

        def wait_out(c, b):
            for j in range(rows_per_chunk):
                pltpu.make_async_copy(
                    rows_v.at[b, pl.ds(j * seq, seq)],
                    out_hbm.at[row_base + c * rows_per_chunk + j],
                    sem_o.at[b]).wait()

        start_gather(0, 0)

        def body(c, carry):
            b = lax.rem(c, 2)
            pb = lax.rem(c - 1, 2)

            @pl.when(c >= 2)
            def _():
                wait_out(c - 2, b)

            start_gather(c, b)
            wait_gather(pb)
            start_out(c - 1, pb)
            return carry

        lax.fori_loop(1, n_chunks, body, 0)

        bl = (n_chunks - 1) % 2
        wait_gather(bl)
        start_out(n_chunks - 1, bl)
        wait_out(n_chunks - 2, 1 - bl)
        wait_out(n_chunks - 1, bl)

    return k(table, idx)


def kernel(token_ids, table):
    bsz, seq = token_ids.shape
    v, d = table.shape
    flat = token_ids.reshape(-1).astype(jnp.int32)
    tflat = lax.optimization_barrier(table.reshape(-1))
    t2 = tflat.reshape(v, d)
    return _gather_rows(flat, t2, (bsz, seq, d))


# speedup vs baseline: 1.5991x; 1.0714x over previous
import functools

import jax
import jax.numpy as jnp
from jax import lax
from jax.experimental import pallas as pl
from jax.experimental.pallas import tpu as pltpu
from jax.experimental.pallas import tpu_sc as plsc

NUM_CORES = 2
NUM_SUBCORES = 16
NUM_WORKERS = NUM_CORES * NUM_SUBCORES
CHUNK = 1600
DETILE_W = 2048


def _detile_table(table_t, v, d):
    w = DETILE_W
    nblk = pl.cdiv(v, w)
    slab = w // 4

    def body(in_ref, out_ref):
        xt = in_ref[...].T
        xtp = jnp.concatenate(
            [xt, jnp.zeros((w, 128 - d), jnp.float32)], axis=1)
        out = xtp[0:slab, :]
        for a in range(1, 4):
            out = out + jnp.roll(xtp[a * slab:(a + 1) * slab, :], a * d, 1)
        out_ref[...] = out

    return pl.pallas_call(
        body,
        grid=(nblk,),
        in_specs=[pl.BlockSpec((d, w), lambda i: (0, i))],
        out_specs=pl.BlockSpec((slab, 128), lambda i: (i, 0)),
        out_shape=jax.ShapeDtypeStruct((nblk * slab, 128), jnp.float32),
    )(table_t)


@functools.partial(jax.jit, static_argnums=(2,))
def _gather_rows(idx, table, out_shape):
    n = idx.shape[0]
    bsz, seq, d = out_shape
    v = table.shape[0]
    rows_per_chunk = CHUNK // seq
    b_per_w = n // NUM_WORKERS
    n_chunks = b_per_w // CHUNK

    tflat = _detile_table(table.T, v, d)
    t2 = tflat.reshape(-1, d)

    mesh = plsc.VectorSubcoreMesh(core_axis_name="c", subcore_axis_name="s")

    @functools.partial(
        pl.kernel,
        mesh=mesh,
        out_type=jax.ShapeDtypeStruct((bsz, seq, d), jnp.float32),
        scratch_types=[
            pltpu.VMEM((2, CHUNK), jnp.int32),
            pltpu.VMEM((2, CHUNK, d), jnp.float32),
            pltpu.SemaphoreType.DMA((2,)),
            pltpu.SemaphoreType.DMA((2,)),
        ],
        compiler_params=pltpu.CompilerParams(use_tc_tiling_on_sc=False),
    )
    def k(table_hbm, idx_hbm, out_hbm, idx_v, rows_v, sem_g, sem_o):
        wid = lax.axis_index("s") * NUM_CORES + lax.axis_index("c")
        base = wid * b_per_w
        row_base = wid * (b_per_w // seq)

        def start_gather(c, b):
            pltpu.sync_copy(idx_hbm.at[pl.ds(base + c * CHUNK, CHUNK)],
                            idx_v.at[b])
            pltpu.make_async_copy(
                table_hbm.at[idx_v.at[b]], rows_v.at[b], sem_g.at[b]).start()

        def wait_gather(b):
            pltpu.make_async_copy(
                table_hbm.at[idx_v.at[b]], rows_v.at[b], sem_g.at[b]).wait()

        def start_out(c, b):
            for j in range(rows_per_chunk):
                pltpu.make_async_copy(
                    rows_v.at[b, pl.ds(j * seq, seq)],
                    out_hbm.at[row_base + c * rows_per_chunk + j],
                    sem_o.at[b]).start()

        def wait_out(c, b):
            for j in range(rows_per_chunk):
                pltpu.make_async_copy(
                    rows_v.at[b, pl.ds(j * seq, seq)],
                    out_hbm.at[row_base + c * rows_per_chunk + j],
                    sem_o.at[b]).wait()

        start_gather(0, 0)

        def body(c, carry):
            b = lax.rem(c, 2)
            pb = lax.rem(c - 1, 2)

            @pl.when(c >= 2)
            def _():
                wait_out(c - 2, b)

            start_gather(c, b)
            wait_gather(pb)
            start_out(c - 1, pb)
            return carry

        lax.fori_loop(1, n_chunks, body, 0)

        bl = (n_chunks - 1) % 2
        wait_gather(bl)
        start_out(n_chunks - 1, bl)
        wait_out(n_chunks - 2, 1 - bl)
        wait_out(n_chunks - 1, bl)

    return k(t2, idx)


def kernel(token_ids, table):
    bsz, seq = token_ids.shape
    v, d = table.shape
    flat = token_ids.reshape(-1).astype(jnp.int32)
    midx = ((flat & ~2047) + ((flat & 511) << 2) + ((flat & 2047) >> 9))
    return _gather_rows(midx, table, (bsz, seq, d))
